# baseline (speedup 1.0000x reference)
import functools

import jax
import jax.numpy as jnp
from jax import lax
from jax.experimental import pallas as pl
from jax.experimental.pallas import tpu as pltpu
from jax.experimental.pallas import tpu_sc as plsc

NC = 2
NS = 16
LW = 128
BN = 1000


def _mesh():
    return plsc.VectorSubcoreMesh(
        core_axis_name="c", subcore_axis_name="s", num_cores=NC,
        num_subcores=NS)


def _deg_body(nt, erp, recv_hbm, out_hbm, deg_sp, ridx_v, ones_v, zslab_v):
    c = lax.axis_index("c")
    s = lax.axis_index("s")
    slab = nt // NS
    rows = erp // (NC * NS)
    ones16 = jnp.ones((16,), jnp.float32)
    zero16 = jnp.zeros((16,), jnp.float32)

    @pl.loop(0, LW // 16)
    def _(i):
        ones_v[pl.ds(i * 16, 16)] = ones16

    @pl.loop(0, slab // 16)
    def _(i):
        zslab_v[pl.ds(i * 16, 16)] = zero16

    pltpu.sync_copy(zslab_v, deg_sp.at[pl.ds(s * slab, slab)])
    pltpu.sync_copy(recv_hbm.at[pl.ds(c * (erp // NC) + s * rows, rows)],
                    ridx_v)
    plsc.subcore_barrier()

    @pl.loop(0, rows)
    def _(j):
        pltpu.sync_copy(ones_v, deg_sp.at[ridx_v.at[j]], add=True)

    plsc.subcore_barrier()
    pltpu.sync_copy(deg_sp.at[pl.ds(s * slab, slab)],
                    out_hbm.at[c, pl.ds(s * slab, slab)])


def _deg_call(rp, nt):
    erp = rp.shape[0]
    body = functools.partial(_deg_body, nt, erp)
    return pl.kernel(
        body,
        out_type=jax.ShapeDtypeStruct((NC, nt), jnp.float32),
        mesh=_mesh(),
        scratch_types=[
            pltpu.VMEM_SHARED((nt,), jnp.float32),
            pltpu.VMEM((erp // (NC * NS), LW), jnp.int32),
            pltpu.VMEM((LW,), jnp.float32),
            pltpu.VMEM((nt // NS,), jnp.float32),
        ],
    )(rp)


def _prep_body(x_ref, w_ref, b_ref, p_ref, hn_ref, hd_ref, invs_ref):
    h = jnp.dot(x_ref[...], w_ref[...],
                preferred_element_type=jnp.float32) + b_ref[...]
    deg = p_ref[:, 0] + p_ref[:, 1] + 1.0
    invs = lax.rsqrt(deg)
    hn = h * invs[:, None]
    hn_ref[0] = hn[:, :128]
    hn_ref[1] = hn[:, 128:]
    hd_ref[...] = h * (1.0 / deg)[:, None]
    invs_ref[...] = invs[:, None]


def _prep_call(x, w, b2, degt, n, nt):
    d = x.shape[1]
    grid = (n // BN,)
    return pl.pallas_call(
        _prep_body,
        grid=grid,
        in_specs=[
            pl.BlockSpec((BN, d), lambda i: (i, 0)),
            pl.BlockSpec((d, d), lambda i: (0, 0)),
            pl.BlockSpec((1, d), lambda i: (0, 0)),
            pl.BlockSpec((BN, 2), lambda i: (i, 0)),
        ],
        out_specs=[
            pl.BlockSpec((2, BN, 128), lambda i: (0, i, 0)),
            pl.BlockSpec((BN, d), lambda i: (i, 0)),
            pl.BlockSpec((BN, 1), lambda i: (i, 0)),
        ],
        out_shape=[
            jax.ShapeDtypeStruct((2, nt, 128), jnp.float32),
            jax.ShapeDtypeStruct((n, d), jnp.float32),
            jax.ShapeDtypeStruct((nt, 1), jnp.float32),
        ],
    )(x, w, b2, degt)


def _hn_body(nt, erp, sidx_hbm, ridx_hbm, hn_hbm, accout_hbm,
             acc_sp, sidx_v, ridx_v, rows_v):
    c = lax.axis_index("c")
    s = lax.axis_index("s")
    rows_hn = erp // NS
    slab = nt // NS
    zero16 = jnp.zeros((16,), jnp.float32)

    @pl.loop(0, LW)
    def _(i):
        for j in range(8):
            rows_v[i, pl.ds(j * 16, 16)] = zero16

    @pl.loop(0, slab // LW)
    def _(i):
        pltpu.sync_copy(rows_v, acc_sp.at[pl.ds(s * slab + i * LW, LW)])

    pltpu.sync_copy(sidx_hbm.at[pl.ds(s * rows_hn, rows_hn)], sidx_v)
    pltpu.sync_copy(ridx_hbm.at[pl.ds(s * rows_hn, rows_hn)], ridx_v)
    plsc.subcore_barrier()

    def hn_loop(table):
        @pl.loop(0, rows_hn)
        def _(j):
            pltpu.sync_copy(table.at[sidx_v.at[j]], rows_v)
            pltpu.sync_copy(rows_v, acc_sp.at[ridx_v.at[j]], add=True)

    pl.when(c == 0)(lambda: hn_loop(hn_hbm.at[0]))
    pl.when(c == 1)(lambda: hn_loop(hn_hbm.at[1]))

    plsc.subcore_barrier()

    @pl.loop(0, slab // LW)
    def _(i):
        r0 = s * slab + i * LW
        pltpu.sync_copy(acc_sp.at[pl.ds(r0, LW)],
                        accout_hbm.at[c, pl.ds(r0, LW)])


def _hn_call(sp, rp, hn2, nt):
    erp = sp.shape[0]
    body = functools.partial(_hn_body, nt, erp)
    return pl.kernel(
        body,
        out_type=jax.ShapeDtypeStruct((NC, nt, 128), jnp.float32),
        mesh=_mesh(),
        scratch_types=[
            pltpu.VMEM_SHARED((nt, 128), jnp.float32),
            pltpu.VMEM((erp // NS, LW), jnp.int32),
            pltpu.VMEM((erp // NS, LW), jnp.int32),
            pltpu.VMEM((LW, 128), jnp.float32),
        ],
    )(sp, rp, hn2)


def _ef_body(nt, erp, sidx_hbm, ridx_hbm, efp_hbm, invsrep_hbm, gout_hbm,
             g_sp, sidxe_v, ridxe_v, efp_v, wsrep_v):
    c = lax.axis_index("c")
    s = lax.axis_index("s")
    wid = s * NC + c
    rows_ef = erp // (NC * NS)
    slab = nt // NS
    zero16 = jnp.zeros((16,), jnp.float32)

    @pl.loop(0, LW)
    def _(i):
        for j in range(2):
            efp_v[i, pl.ds(j * 16, 16)] = zero16

    @pl.loop(0, slab // LW)
    def _(i):
        pltpu.sync_copy(efp_v, g_sp.at[pl.ds(s * slab + i * LW, LW)])

    pltpu.sync_copy(sidx_hbm.at[pl.ds(wid * rows_ef, rows_ef)], sidxe_v)
    pltpu.sync_copy(ridx_hbm.at[pl.ds(wid * rows_ef, rows_ef)], ridxe_v)
    plsc.subcore_barrier()

    @pl.loop(0, rows_ef)
    def _(j):
        pltpu.sync_copy(efp_hbm.at[pl.ds((wid * rows_ef + j) * LW, LW)],
                        efp_v)
        pltpu.sync_copy(invsrep_hbm.at[sidxe_v.at[j]], wsrep_v)

        @pl.loop(0, LW)
        def _(i):
            w16 = wsrep_v[i, :]
            efp_v[i, pl.ds(0, 16)] = efp_v[i, pl.ds(0, 16)] * w16
            efp_v[i, pl.ds(16, 16)] = efp_v[i, pl.ds(16, 16)] * w16

        pltpu.sync_copy(efp_v, g_sp.at[ridxe_v.at[j]], add=True)

    plsc.subcore_barrier()

    @pl.loop(0, slab // LW)
    def _(i):
        r0 = s * slab + i * LW
        pltpu.sync_copy(g_sp.at[pl.ds(r0, LW)],
                        gout_hbm.at[c, pl.ds(r0, LW)])


def _ef_call(sp, rp, efp, invsrep, nt):
    erp = sp.shape[0]
    body = functools.partial(_ef_body, nt, erp)
    return pl.kernel(
        body,
        out_type=jax.ShapeDtypeStruct((NC, nt, 32), jnp.float32),
        mesh=_mesh(),
        compiler_params=pltpu.CompilerParams(use_tc_tiling_on_sc=False),
        scratch_types=[
            pltpu.VMEM_SHARED((nt, 32), jnp.float32),
            pltpu.VMEM((erp // (NC * NS), LW), jnp.int32),
            pltpu.VMEM((erp // (NC * NS), LW), jnp.int32),
            pltpu.VMEM((LW, 32), jnp.float32),
            pltpu.VMEM((LW, 16), jnp.float32),
        ],
    )(sp, rp, efp, invsrep)


def _final_body(acc_ref, g_ref, hd_ref, invs_ref, waug_ref, out_ref):
    g = g_ref[0] + g_ref[1]
    gc = jnp.dot(g, waug_ref[...], preferred_element_type=jnp.float32)
    acc = jnp.concatenate([acc_ref[0], acc_ref[1]], axis=1)
    out_ref[...] = invs_ref[...] * (acc + gc) + hd_ref[...]


def _final_call(acc2, g2, hd, invs1, waug, n, nt):
    d = hd.shape[1]
    grid = (n // BN,)
    return pl.pallas_call(
        _final_body,
        grid=grid,
        in_specs=[
            pl.BlockSpec((2, BN, 128), lambda i: (0, i, 0)),
            pl.BlockSpec((2, BN, 32), lambda i: (0, i, 0)),
            pl.BlockSpec((BN, d), lambda i: (i, 0)),
            pl.BlockSpec((BN, 1), lambda i: (i, 0)),
            pl.BlockSpec((32, d), lambda i: (0, 0)),
        ],
        out_specs=pl.BlockSpec((BN, d), lambda i: (i, 0)),
        out_shape=jax.ShapeDtypeStruct((n, d), jnp.float32),
    )(acc2, g2, hd, invs1, waug)


def kernel(node_features, senders, receivers, edge_features, W_kernel,
           W_bias, We_kernel, We_bias):
    n, d = node_features.shape
    e, de = edge_features.shape

    nt = (-(-n // 128)) * 128 + 128
    erows = -(-e // LW)
    erp = -(-erows // (NC * NS)) * (NC * NS)
    ep = erp * LW
    npad = nt - n

    s32 = senders.astype(jnp.int32)
    r32 = receivers.astype(jnp.int32)
    pad_idx = n + (jnp.arange(ep - e, dtype=jnp.int32) % npad)
    sp = jnp.concatenate([s32, pad_idx]).reshape(erp, LW)
    rp = jnp.concatenate([r32, pad_idx]).reshape(erp, LW)

    ef32 = edge_features.astype(jnp.float32)
    efp = jnp.concatenate(
        [ef32, jnp.ones((e, 1), jnp.float32),
         jnp.zeros((e, 15), jnp.float32)], axis=1)
    efp = jnp.concatenate([efp, jnp.zeros((ep - e, 32), jnp.float32)],
                          axis=0)
    waug = jnp.concatenate(
        [We_kernel.astype(jnp.float32),
         We_bias.astype(jnp.float32)[None, :],
         jnp.zeros((15, d), jnp.float32)], axis=0)

    degp = _deg_call(rp, nt)
    degt = jnp.transpose(degp)
    hn2, hd, invs1 = _prep_call(
        node_features.astype(jnp.float32), W_kernel.astype(jnp.float32),
        W_bias.astype(jnp.float32).reshape(1, d), degt, n, nt)
    invsrep = jnp.broadcast_to(invs1, (nt, 16))
    acc2 = _hn_call(sp, rp, hn2, nt)
    g2 = _ef_call(sp, rp, efp, invsrep, nt)
    return _final_call(acc2, g2, hd, invs1, waug, n, nt)


# speedup vs baseline: 1.1664x; 1.1664x over previous
import functools

import jax
import jax.numpy as jnp
from jax import lax
from jax.experimental import pallas as pl
from jax.experimental.pallas import tpu as pltpu
from jax.experimental.pallas import tpu_sc as plsc

NC = 2
NS = 16
LW = 128
BN = 1000


def _mesh():
    return plsc.VectorSubcoreMesh(
        core_axis_name="c", subcore_axis_name="s", num_cores=NC,
        num_subcores=NS)


def _deg_body(nt, erp, recv_hbm, out_hbm, deg_sp, ridx_v, ones_v, zslab_v):
    c = lax.axis_index("c")
    s = lax.axis_index("s")
    slab = nt // NS
    rows = erp // (NC * NS)
    ones16 = jnp.ones((16,), jnp.float32)
    zero16 = jnp.zeros((16,), jnp.float32)

    @pl.loop(0, LW // 16)
    def _(i):
        ones_v[pl.ds(i * 16, 16)] = ones16

    @pl.loop(0, slab // 16)
    def _(i):
        zslab_v[pl.ds(i * 16, 16)] = zero16

    pltpu.sync_copy(zslab_v, deg_sp.at[pl.ds(s * slab, slab)])
    pltpu.sync_copy(recv_hbm.at[pl.ds(c * (erp // NC) + s * rows, rows)],
                    ridx_v)
    plsc.subcore_barrier()

    @pl.loop(0, rows)
    def _(j):
        pltpu.sync_copy(ones_v, deg_sp.at[ridx_v.at[j]], add=True)

    plsc.subcore_barrier()
    pltpu.sync_copy(deg_sp.at[pl.ds(s * slab, slab)],
                    out_hbm.at[c, pl.ds(s * slab, slab)])


def _deg_call(rp, nt):
    erp = rp.shape[0]
    body = functools.partial(_deg_body, nt, erp)
    return pl.kernel(
        body,
        out_type=jax.ShapeDtypeStruct((NC, nt), jnp.float32),
        mesh=_mesh(),
        scratch_types=[
            pltpu.VMEM_SHARED((nt,), jnp.float32),
            pltpu.VMEM((erp // (NC * NS), LW), jnp.int32),
            pltpu.VMEM((LW,), jnp.float32),
            pltpu.VMEM((nt // NS,), jnp.float32),
        ],
    )(rp)


def _prep_body(x_ref, w_ref, b_ref, p_ref, hn_ref, hd_ref, invs_ref):
    h = jnp.dot(x_ref[...], w_ref[...],
                preferred_element_type=jnp.float32) + b_ref[...]
    deg = p_ref[:, 0] + p_ref[:, 1] + 1.0
    invs = lax.rsqrt(deg)
    hn = h * invs[:, None]
    hn_ref[0] = hn[:, :128]
    hn_ref[1] = hn[:, 128:]
    hd_ref[...] = h * (1.0 / deg)[:, None]
    invs_ref[...] = invs[:, None]


def _prep_call(x, w, b2, degt, n, nt):
    d = x.shape[1]
    grid = (n // BN,)
    return pl.pallas_call(
        _prep_body,
        grid=grid,
        in_specs=[
            pl.BlockSpec((BN, d), lambda i: (i, 0)),
            pl.BlockSpec((d, d), lambda i: (0, 0)),
            pl.BlockSpec((1, d), lambda i: (0, 0)),
            pl.BlockSpec((BN, 2), lambda i: (i, 0)),
        ],
        out_specs=[
            pl.BlockSpec((2, BN, 128), lambda i: (0, i, 0)),
            pl.BlockSpec((BN, d), lambda i: (i, 0)),
            pl.BlockSpec((BN, 1), lambda i: (i, 0)),
        ],
        out_shape=[
            jax.ShapeDtypeStruct((2, nt, 128), jnp.float32),
            jax.ShapeDtypeStruct((n, d), jnp.float32),
            jax.ShapeDtypeStruct((nt, 1), jnp.float32),
        ],
    )(x, w, b2, degt)


NB = 8


def _hn_body(nt, erp, sidx_hbm, ridx_hbm, hn_hbm, accout_hbm,
             acc_sp, sidx_v, ridx_ring, rows_a, rows_b, sema, semb, semr):
    c = lax.axis_index("c")
    s = lax.axis_index("s")
    rows_hn = erp // NS
    slab = nt // NS
    nblk = rows_hn // NB
    zero16 = jnp.zeros((16,), jnp.float32)

    @pl.loop(0, LW)
    def _(i):
        for j in range(8):
            rows_a[i, pl.ds(j * 16, 16)] = zero16

    @pl.loop(0, slab // LW)
    def _(i):
        pltpu.sync_copy(rows_a, acc_sp.at[pl.ds(s * slab + i * LW, LW)])

    base = s * rows_hn
    pltpu.sync_copy(sidx_hbm.at[pl.ds(base, rows_hn)], sidx_v)
    plsc.subcore_barrier()

    def hn_loop(table):
        pltpu.async_copy(ridx_hbm.at[pl.ds(base, NB)], ridx_ring.at[0],
                         semr)
        pltpu.async_copy(table.at[sidx_v.at[0]], rows_a, sema)
        pltpu.async_copy(table.at[sidx_v.at[1]], rows_b, semb)

        @pl.loop(0, nblk)
        def _(g):
            p = lax.rem(g, 2)
            pltpu.make_async_copy(ridx_hbm.at[pl.ds(base, NB)],
                                  ridx_ring.at[0], semr).wait()

            @pl.when(g < nblk - 1)
            def _():
                pltpu.async_copy(
                    ridx_hbm.at[pl.ds(base + (g + 1) * NB, NB)],
                    ridx_ring.at[1 - p], semr)

            @pl.loop(0, NB // 2)
            def _(t):
                j = g * NB + 2 * t
                pltpu.make_async_copy(table.at[sidx_v.at[0]], rows_a,
                                      sema).wait()
                pltpu.sync_copy(rows_a, acc_sp.at[ridx_ring.at[p, 2 * t]],
                                add=True)

                @pl.when(j + 2 < rows_hn)
                def _():
                    pltpu.async_copy(table.at[sidx_v.at[j + 2]], rows_a,
                                     sema)

                pltpu.make_async_copy(table.at[sidx_v.at[0]], rows_b,
                                      semb).wait()
                pltpu.sync_copy(rows_b,
                                acc_sp.at[ridx_ring.at[p, 2 * t + 1]],
                                add=True)

                @pl.when(j + 3 < rows_hn)
                def _():
                    pltpu.async_copy(table.at[sidx_v.at[j + 3]], rows_b,
                                     semb)

    pl.when(c == 0)(lambda: hn_loop(hn_hbm.at[0]))
    pl.when(c == 1)(lambda: hn_loop(hn_hbm.at[1]))

    plsc.subcore_barrier()

    @pl.loop(0, slab // LW)
    def _(i):
        r0 = s * slab + i * LW
        pltpu.sync_copy(acc_sp.at[pl.ds(r0, LW)],
                        accout_hbm.at[c, pl.ds(r0, LW)])


def _hn_call(sp, rp, hn2, nt):
    erp = sp.shape[0]
    body = functools.partial(_hn_body, nt, erp)
    return pl.kernel(
        body,
        out_type=jax.ShapeDtypeStruct((NC, nt, 128), jnp.float32),
        mesh=_mesh(),
        scratch_types=[
            pltpu.VMEM_SHARED((nt, 128), jnp.float32),
            pltpu.VMEM((erp // NS, LW), jnp.int32),
            pltpu.VMEM((2, NB, LW), jnp.int32),
            pltpu.VMEM((LW, 128), jnp.float32),
            pltpu.VMEM((LW, 128), jnp.float32),
            pltpu.SemaphoreType.DMA,
            pltpu.SemaphoreType.DMA,
            pltpu.SemaphoreType.DMA,
        ],
    )(sp, rp, hn2)


def _ef_body(nt, erp, sidx_hbm, ridx_hbm, efp_hbm, invsrep_hbm, gout_hbm,
             g_sp, sidxe_v, ridxe_v, efp_v, wsrep_v):
    c = lax.axis_index("c")
    s = lax.axis_index("s")
    wid = s * NC + c
    rows_ef = erp // (NC * NS)
    slab = nt // NS
    zero16 = jnp.zeros((16,), jnp.float32)

    @pl.loop(0, LW)
    def _(i):
        for j in range(2):
            efp_v[i, pl.ds(j * 16, 16)] = zero16

    @pl.loop(0, slab // LW)
    def _(i):
        pltpu.sync_copy(efp_v, g_sp.at[pl.ds(s * slab + i * LW, LW)])

    pltpu.sync_copy(sidx_hbm.at[pl.ds(wid * rows_ef, rows_ef)], sidxe_v)
    pltpu.sync_copy(ridx_hbm.at[pl.ds(wid * rows_ef, rows_ef)], ridxe_v)
    plsc.subcore_barrier()

    @pl.loop(0, rows_ef)
    def _(j):
        pltpu.sync_copy(efp_hbm.at[pl.ds((wid * rows_ef + j) * LW, LW)],
                        efp_v)
        pltpu.sync_copy(invsrep_hbm.at[sidxe_v.at[j]], wsrep_v)

        @pl.loop(0, LW)
        def _(i):
            w16 = wsrep_v[i, :]
            efp_v[i, pl.ds(0, 16)] = efp_v[i, pl.ds(0, 16)] * w16
            efp_v[i, pl.ds(16, 16)] = efp_v[i, pl.ds(16, 16)] * w16

        pltpu.sync_copy(efp_v, g_sp.at[ridxe_v.at[j]], add=True)

    plsc.subcore_barrier()

    @pl.loop(0, slab // LW)
    def _(i):
        r0 = s * slab + i * LW
        pltpu.sync_copy(g_sp.at[pl.ds(r0, LW)],
                        gout_hbm.at[c, pl.ds(r0, LW)])


def _ef_call(sp, rp, efp, invsrep, nt):
    erp = sp.shape[0]
    body = functools.partial(_ef_body, nt, erp)
    return pl.kernel(
        body,
        out_type=jax.ShapeDtypeStruct((NC, nt, 32), jnp.float32),
        mesh=_mesh(),
        compiler_params=pltpu.CompilerParams(use_tc_tiling_on_sc=False),
        scratch_types=[
            pltpu.VMEM_SHARED((nt, 32), jnp.float32),
            pltpu.VMEM((erp // (NC * NS), LW), jnp.int32),
            pltpu.VMEM((erp // (NC * NS), LW), jnp.int32),
            pltpu.VMEM((LW, 32), jnp.float32),
            pltpu.VMEM((LW, 16), jnp.float32),
        ],
    )(sp, rp, efp, invsrep)


def _final_body(acc_ref, g_ref, hd_ref, invs_ref, waug_ref, out_ref):
    g = g_ref[0] + g_ref[1]
    gc = jnp.dot(g, waug_ref[...], preferred_element_type=jnp.float32)
    acc = jnp.concatenate([acc_ref[0], acc_ref[1]], axis=1)
    out_ref[...] = invs_ref[...] * (acc + gc) + hd_ref[...]


def _final_call(acc2, g2, hd, invs1, waug, n, nt):
    d = hd.shape[1]
    grid = (n // BN,)
    return pl.pallas_call(
        _final_body,
        grid=grid,
        in_specs=[
            pl.BlockSpec((2, BN, 128), lambda i: (0, i, 0)),
            pl.BlockSpec((2, BN, 32), lambda i: (0, i, 0)),
            pl.BlockSpec((BN, d), lambda i: (i, 0)),
            pl.BlockSpec((BN, 1), lambda i: (i, 0)),
            pl.BlockSpec((32, d), lambda i: (0, 0)),
        ],
        out_specs=pl.BlockSpec((BN, d), lambda i: (i, 0)),
        out_shape=jax.ShapeDtypeStruct((n, d), jnp.float32),
    )(acc2, g2, hd, invs1, waug)


def kernel(node_features, senders, receivers, edge_features, W_kernel,
           W_bias, We_kernel, We_bias):
    n, d = node_features.shape
    e, de = edge_features.shape

    nt = (-(-n // 128)) * 128 + 128
    erows = -(-e // LW)
    erp = -(-erows // (NC * NS)) * (NC * NS)
    ep = erp * LW
    npad = nt - n

    s32 = senders.astype(jnp.int32)
    r32 = receivers.astype(jnp.int32)
    pad_idx = n + (jnp.arange(ep - e, dtype=jnp.int32) % npad)
    sp = jnp.concatenate([s32, pad_idx]).reshape(erp, LW)
    rp = jnp.concatenate([r32, pad_idx]).reshape(erp, LW)

    ef32 = edge_features.astype(jnp.float32)
    efp = jnp.concatenate(
        [ef32, jnp.ones((e, 1), jnp.float32),
         jnp.zeros((e, 15), jnp.float32)], axis=1)
    efp = jnp.concatenate([efp, jnp.zeros((ep - e, 32), jnp.float32)],
                          axis=0)
    waug = jnp.concatenate(
        [We_kernel.astype(jnp.float32),
         We_bias.astype(jnp.float32)[None, :],
         jnp.zeros((15, d), jnp.float32)], axis=0)

    degp = _deg_call(rp, nt)
    degt = jnp.transpose(degp)
    hn2, hd, invs1 = _prep_call(
        node_features.astype(jnp.float32), W_kernel.astype(jnp.float32),
        W_bias.astype(jnp.float32).reshape(1, d), degt, n, nt)
    invsrep = jnp.broadcast_to(invs1, (nt, 16))
    acc2 = _hn_call(sp, rp, hn2, nt)
    g2 = _ef_call(sp, rp, efp, invsrep, nt)
    return _final_call(acc2, g2, hd, invs1, waug, n, nt)


# speedup vs baseline: 1.3786x; 1.1820x over previous
import functools

import jax
import jax.numpy as jnp
from jax import lax
from jax.experimental import pallas as pl
from jax.experimental.pallas import tpu as pltpu
from jax.experimental.pallas import tpu_sc as plsc

NC = 2
NS = 16
LW = 128
BN = 1000


def _mesh():
    return plsc.VectorSubcoreMesh(
        core_axis_name="c", subcore_axis_name="s", num_cores=NC,
        num_subcores=NS)


def _deg_body(nt, erp, recv_hbm, out_hbm, deg_sp, ridx_v, ones_v, zslab_v):
    c = lax.axis_index("c")
    s = lax.axis_index("s")
    slab = nt // NS
    rows = erp // (NC * NS)
    ones16 = jnp.ones((16,), jnp.float32)
    zero16 = jnp.zeros((16,), jnp.float32)

    @pl.loop(0, LW // 16)
    def _(i):
        ones_v[pl.ds(i * 16, 16)] = ones16

    @pl.loop(0, slab // 16)
    def _(i):
        zslab_v[pl.ds(i * 16, 16)] = zero16

    pltpu.sync_copy(zslab_v, deg_sp.at[pl.ds(s * slab, slab)])
    pltpu.sync_copy(recv_hbm.at[pl.ds(c * (erp // NC) + s * rows, rows)],
                    ridx_v)
    plsc.subcore_barrier()

    @pl.loop(0, rows)
    def _(j):
        pltpu.sync_copy(ones_v, deg_sp.at[ridx_v.at[j]], add=True)

    plsc.subcore_barrier()
    pltpu.sync_copy(deg_sp.at[pl.ds(s * slab, slab)],
                    out_hbm.at[c, pl.ds(s * slab, slab)])


def _deg_call(rp, nt):
    erp = rp.shape[0]
    body = functools.partial(_deg_body, nt, erp)
    return pl.kernel(
        body,
        out_type=jax.ShapeDtypeStruct((NC, nt), jnp.float32),
        mesh=_mesh(),
        scratch_types=[
            pltpu.VMEM_SHARED((nt,), jnp.float32),
            pltpu.VMEM((erp // (NC * NS), LW), jnp.int32),
            pltpu.VMEM((LW,), jnp.float32),
            pltpu.VMEM((nt // NS,), jnp.float32),
        ],
    )(rp)


def _prep_body(x_ref, w_ref, b_ref, p_ref, hn_ref, hd_ref, invs_ref):
    h = jnp.dot(x_ref[...], w_ref[...],
                preferred_element_type=jnp.float32) + b_ref[...]
    deg = p_ref[:, 0] + p_ref[:, 1] + 1.0
    invs = lax.rsqrt(deg)
    hn = h * invs[:, None]
    hn_ref[0] = hn[:, :128]
    hn_ref[1] = hn[:, 128:]
    hd_ref[...] = h * (1.0 / deg)[:, None]
    invs_ref[...] = invs[:, None]


def _prep_call(x, w, b2, degt, n, nt):
    d = x.shape[1]
    grid = (n // BN,)
    return pl.pallas_call(
        _prep_body,
        grid=grid,
        in_specs=[
            pl.BlockSpec((BN, d), lambda i: (i, 0)),
            pl.BlockSpec((d, d), lambda i: (0, 0)),
            pl.BlockSpec((1, d), lambda i: (0, 0)),
            pl.BlockSpec((BN, 2), lambda i: (i, 0)),
        ],
        out_specs=[
            pl.BlockSpec((2, BN, 128), lambda i: (0, i, 0)),
            pl.BlockSpec((BN, d), lambda i: (i, 0)),
            pl.BlockSpec((BN, 1), lambda i: (i, 0)),
        ],
        out_shape=[
            jax.ShapeDtypeStruct((2, nt, 128), jnp.float32),
            jax.ShapeDtypeStruct((n, d), jnp.float32),
            jax.ShapeDtypeStruct((nt, 1), jnp.float32),
        ],
    )(x, w, b2, degt)


NB = 8


def _hn_body(nt, erp, sidx_hbm, ridx_hbm, hn_hbm, accout_hbm,
             acc_sp, sidx_v, ridx_ring, rows_a, rows_b, sema, semb, semr):
    c = lax.axis_index("c")
    s = lax.axis_index("s")
    rows_hn = erp // NS
    slab = nt // NS
    nblk = rows_hn // NB
    zero16 = jnp.zeros((16,), jnp.float32)

    @pl.loop(0, LW)
    def _(i):
        for j in range(8):
            rows_a[i, pl.ds(j * 16, 16)] = zero16

    @pl.loop(0, slab // LW)
    def _(i):
        pltpu.sync_copy(rows_a, acc_sp.at[pl.ds(s * slab + i * LW, LW)])

    base = s * rows_hn
    pltpu.sync_copy(sidx_hbm.at[pl.ds(base, rows_hn)], sidx_v)
    plsc.subcore_barrier()

    def hn_loop(table):
        pltpu.async_copy(ridx_hbm.at[pl.ds(base, NB)], ridx_ring.at[0],
                         semr)
        pltpu.async_copy(table.at[sidx_v.at[0]], rows_a, sema)
        pltpu.async_copy(table.at[sidx_v.at[1]], rows_b, semb)

        @pl.loop(0, nblk)
        def _(g):
            p = lax.rem(g, 2)
            pltpu.make_async_copy(ridx_hbm.at[pl.ds(base, NB)],
                                  ridx_ring.at[0], semr).wait()

            @pl.when(g < nblk - 1)
            def _():
                pltpu.async_copy(
                    ridx_hbm.at[pl.ds(base + (g + 1) * NB, NB)],
                    ridx_ring.at[1 - p], semr)

            @pl.loop(0, NB // 2)
            def _(t):
                j = g * NB + 2 * t
                pltpu.make_async_copy(table.at[sidx_v.at[0]], rows_a,
                                      sema).wait()
                pltpu.sync_copy(rows_a, acc_sp.at[ridx_ring.at[p, 2 * t]],
                                add=True)

                @pl.when(j + 2 < rows_hn)
                def _():
                    pltpu.async_copy(table.at[sidx_v.at[j + 2]], rows_a,
                                     sema)

                pltpu.make_async_copy(table.at[sidx_v.at[0]], rows_b,
                                      semb).wait()
                pltpu.sync_copy(rows_b,
                                acc_sp.at[ridx_ring.at[p, 2 * t + 1]],
                                add=True)

                @pl.when(j + 3 < rows_hn)
                def _():
                    pltpu.async_copy(table.at[sidx_v.at[j + 3]], rows_b,
                                     semb)

    pl.when(c == 0)(lambda: hn_loop(hn_hbm.at[0]))
    pl.when(c == 1)(lambda: hn_loop(hn_hbm.at[1]))

    plsc.subcore_barrier()

    @pl.loop(0, slab // LW)
    def _(i):
        r0 = s * slab + i * LW
        pltpu.sync_copy(acc_sp.at[pl.ds(r0, LW)],
                        accout_hbm.at[c, pl.ds(r0, LW)])


def _hn_call(sp, rp, hn2, nt):
    erp = sp.shape[0]
    body = functools.partial(_hn_body, nt, erp)
    return pl.kernel(
        body,
        out_type=jax.ShapeDtypeStruct((NC, nt, 128), jnp.float32),
        mesh=_mesh(),
        scratch_types=[
            pltpu.VMEM_SHARED((nt, 128), jnp.float32),
            pltpu.VMEM((erp // NS, LW), jnp.int32),
            pltpu.VMEM((2, NB, LW), jnp.int32),
            pltpu.VMEM((LW, 128), jnp.float32),
            pltpu.VMEM((LW, 128), jnp.float32),
            pltpu.SemaphoreType.DMA,
            pltpu.SemaphoreType.DMA,
            pltpu.SemaphoreType.DMA,
        ],
    )(sp, rp, hn2)


def _ef_body(nt, erp, sidx_hbm, ridx_hbm, efp_hbm, invsrep_hbm, gout_hbm,
             g_sp, sidxe_v, ridxe_v, efp_a, efp_b, ws_a, ws_b,
             sea, seb, swa, swb):
    c = lax.axis_index("c")
    s = lax.axis_index("s")
    wid = s * NC + c
    rows_ef = erp // (NC * NS)
    slab = nt // NS
    zero16 = jnp.zeros((16,), jnp.float32)

    @pl.loop(0, LW)
    def _(i):
        for j in range(2):
            efp_a[i, pl.ds(j * 16, 16)] = zero16

    @pl.loop(0, slab // LW)
    def _(i):
        pltpu.sync_copy(efp_a, g_sp.at[pl.ds(s * slab + i * LW, LW)])

    ebase = wid * rows_ef
    pltpu.sync_copy(sidx_hbm.at[pl.ds(ebase, rows_ef)], sidxe_v)
    pltpu.sync_copy(ridx_hbm.at[pl.ds(ebase, rows_ef)], ridxe_v)
    plsc.subcore_barrier()

    def stage(j, efp_v, ws_v, se, sw):
        pltpu.async_copy(efp_hbm.at[pl.ds((ebase + j) * LW, LW)], efp_v,
                         se)
        pltpu.async_copy(invsrep_hbm.at[sidxe_v.at[j]], ws_v, sw)

    def run(j, efp_v, ws_v, se, sw):
        pltpu.make_async_copy(efp_hbm.at[pl.ds(0, LW)], efp_v, se).wait()
        pltpu.make_async_copy(invsrep_hbm.at[sidxe_v.at[0]], ws_v,
                              sw).wait()

        @pl.loop(0, LW, unroll=4)
        def _(i):
            w16 = ws_v[i, :]
            efp_v[i, pl.ds(0, 16)] = efp_v[i, pl.ds(0, 16)] * w16
            efp_v[i, pl.ds(16, 16)] = efp_v[i, pl.ds(16, 16)] * w16

        pltpu.sync_copy(efp_v, g_sp.at[ridxe_v.at[j]], add=True)

        @pl.when(j + 2 < rows_ef)
        def _():
            stage(j + 2, efp_v, ws_v, se, sw)

    stage(0, efp_a, ws_a, sea, swa)
    stage(1, efp_b, ws_b, seb, swb)

    @pl.loop(0, rows_ef // 2)
    def _(k):
        run(2 * k, efp_a, ws_a, sea, swa)
        run(2 * k + 1, efp_b, ws_b, seb, swb)

    plsc.subcore_barrier()

    @pl.loop(0, slab // LW)
    def _(i):
        r0 = s * slab + i * LW
        pltpu.sync_copy(g_sp.at[pl.ds(r0, LW)],
                        gout_hbm.at[c, pl.ds(r0, LW)])


def _ef_call(sp, rp, efp, invsrep, nt):
    erp = sp.shape[0]
    body = functools.partial(_ef_body, nt, erp)
    return pl.kernel(
        body,
        out_type=jax.ShapeDtypeStruct((NC, nt, 32), jnp.float32),
        mesh=_mesh(),
        compiler_params=pltpu.CompilerParams(use_tc_tiling_on_sc=False),
        scratch_types=[
            pltpu.VMEM_SHARED((nt, 32), jnp.float32),
            pltpu.VMEM((erp // (NC * NS), LW), jnp.int32),
            pltpu.VMEM((erp // (NC * NS), LW), jnp.int32),
            pltpu.VMEM((LW, 32), jnp.float32),
            pltpu.VMEM((LW, 32), jnp.float32),
            pltpu.VMEM((LW, 16), jnp.float32),
            pltpu.VMEM((LW, 16), jnp.float32),
            pltpu.SemaphoreType.DMA,
            pltpu.SemaphoreType.DMA,
            pltpu.SemaphoreType.DMA,
            pltpu.SemaphoreType.DMA,
        ],
    )(sp, rp, efp, invsrep)


def _final_body(acc_ref, g_ref, hd_ref, invs_ref, waug_ref, out_ref):
    g = g_ref[0] + g_ref[1]
    gc = jnp.dot(g, waug_ref[...], preferred_element_type=jnp.float32)
    acc = jnp.concatenate([acc_ref[0], acc_ref[1]], axis=1)
    out_ref[...] = invs_ref[...] * (acc + gc) + hd_ref[...]


def _final_call(acc2, g2, hd, invs1, waug, n, nt):
    d = hd.shape[1]
    grid = (n // BN,)
    return pl.pallas_call(
        _final_body,
        grid=grid,
        in_specs=[
            pl.BlockSpec((2, BN, 128), lambda i: (0, i, 0)),
            pl.BlockSpec((2, BN, 32), lambda i: (0, i, 0)),
            pl.BlockSpec((BN, d), lambda i: (i, 0)),
            pl.BlockSpec((BN, 1), lambda i: (i, 0)),
            pl.BlockSpec((32, d), lambda i: (0, 0)),
        ],
        out_specs=pl.BlockSpec((BN, d), lambda i: (i, 0)),
        out_shape=jax.ShapeDtypeStruct((n, d), jnp.float32),
    )(acc2, g2, hd, invs1, waug)


def kernel(node_features, senders, receivers, edge_features, W_kernel,
           W_bias, We_kernel, We_bias):
    n, d = node_features.shape
    e, de = edge_features.shape

    nt = (-(-n // 128)) * 128 + 128
    erows = -(-e // LW)
    erp = -(-erows // (NC * NS)) * (NC * NS)
    ep = erp * LW
    npad = nt - n

    s32 = senders.astype(jnp.int32)
    r32 = receivers.astype(jnp.int32)
    pad_idx = n + (jnp.arange(ep - e, dtype=jnp.int32) % npad)
    sp = jnp.concatenate([s32, pad_idx]).reshape(erp, LW)
    rp = jnp.concatenate([r32, pad_idx]).reshape(erp, LW)

    ef32 = edge_features.astype(jnp.float32)
    efp = jnp.concatenate(
        [ef32, jnp.ones((e, 1), jnp.float32),
         jnp.zeros((e, 15), jnp.float32)], axis=1)
    efp = jnp.concatenate([efp, jnp.zeros((ep - e, 32), jnp.float32)],
                          axis=0)
    waug = jnp.concatenate(
        [We_kernel.astype(jnp.float32),
         We_bias.astype(jnp.float32)[None, :],
         jnp.zeros((15, d), jnp.float32)], axis=0)

    degp = _deg_call(rp, nt)
    degt = jnp.transpose(degp)
    hn2, hd, invs1 = _prep_call(
        node_features.astype(jnp.float32), W_kernel.astype(jnp.float32),
        W_bias.astype(jnp.float32).reshape(1, d), degt, n, nt)
    invsrep = jnp.broadcast_to(invs1, (nt, 16))
    acc2 = _hn_call(sp, rp, hn2, nt)
    g2 = _ef_call(sp, rp, efp, invsrep, nt)
    return _final_call(acc2, g2, hd, invs1, waug, n, nt)
